# baseline (speedup 1.0000x reference)
import functools

import jax
import jax.numpy as jnp
from jax import lax
from jax.experimental import pallas as pl
from jax.experimental.pallas import tpu as pltpu
from jax.experimental.pallas import tpu_sc as plsc

_B = 4096
_L = 50
_LP = 56
_D = 128
_H1 = 256
_H2 = 128
_NCLS = 50
_EPS = 1e-5

_NC = 2
_NS = 16
_NW = _NC * _NS
_EPW = _B // _NW
_PAIRS = _EPW // 2
_PW = 2 * _LP
_LANES = 16
_VPR = _D // _LANES


_NBUF = 8


def _sc_pool_body(tok_hbm, emb_hbm, out_hbm, tok_v, rows, pooled_v, *sems):
    c = lax.axis_index("c")
    s = lax.axis_index("s")
    w = c * _NS + s

    pltpu.sync_copy(tok_hbm.at[pl.ds(w * _EPW, _EPW)], tok_v)

    def fire(i, b):
        pltpu.make_async_copy(
            emb_hbm.at[tok_v.at[i]], rows.at[b], sems[b]
        ).start()

    def wait(b):
        pltpu.make_async_copy(
            emb_hbm.at[pl.ds(0, _LP)], rows.at[b], sems[b]
        ).wait()

    for b in range(_NBUF):
        fire(b, b)

    def step(k, carry):
        i0 = k * _NBUF
        for b in range(_NBUF):
            i = i0 + b
            wait(b)

            buf = rows.at[b]

            def acc_body(r, acc, buf=buf):
                return tuple(
                    acc[v] + buf[r, pl.ds(v * _LANES, _LANES)]
                    for v in range(_VPR)
                )

            acc = lax.fori_loop(
                0, _LP, acc_body,
                tuple(jnp.zeros((_LANES,), jnp.float32)
                      for _ in range(_VPR)),
            )
            for v in range(_VPR):
                pooled_v[i, pl.ds(v * _LANES, _LANES)] = acc[v]

            @pl.when(i + _NBUF < _EPW)
            def _():
                fire(i + _NBUF, b)
        return carry

    lax.fori_loop(0, _EPW // _NBUF, step, 0)

    pltpu.sync_copy(pooled_v, out_hbm.at[pl.ds(w * _EPW, _EPW)])


_sc_pool = functools.partial(
    pl.kernel,
    out_type=jax.ShapeDtypeStruct((_B, _D), jnp.float32),
    mesh=plsc.VectorSubcoreMesh(core_axis_name="c", subcore_axis_name="s"),
    scratch_types=[
        pltpu.VMEM((_EPW, _LP), jnp.int32),
        pltpu.VMEM((_NBUF, _LP, _D), jnp.float32),
        pltpu.VMEM((_EPW, _D), jnp.float32),
    ] + [pltpu.SemaphoreType.DMA] * _NBUF,
)(_sc_pool_body)


def _mlp_body(ps, w1, b1, g1, be1, w2, b2, g2, be2, wout, bout, out):
    x = ps[...] * (1.0 / _L)

    h = lax.dot_general(x, w1[...], (((1,), (1,)), ((), ())),
                        preferred_element_type=jnp.float32) + b1[...]
    mu = jnp.mean(h, axis=0, keepdims=True)
    d = h - mu
    var = jnp.mean(d * d, axis=0, keepdims=True)
    h = g1[...] * d / jnp.sqrt(var + _EPS) + be1[...]
    h = jnp.maximum(h, 0.0)

    h = lax.dot_general(h, w2[...], (((1,), (1,)), ((), ())),
                        preferred_element_type=jnp.float32) + b2[...]
    mu = jnp.mean(h, axis=0, keepdims=True)
    d = h - mu
    var = jnp.mean(d * d, axis=0, keepdims=True)
    h = g2[...] * d / jnp.sqrt(var + _EPS) + be2[...]
    h = jnp.maximum(h, 0.0)

    out[...] = lax.dot_general(h, wout[...], (((1,), (1,)), ((), ())),
                               preferred_element_type=jnp.float32) + bout[...]


_mlp = pl.pallas_call(
    _mlp_body,
    out_shape=jax.ShapeDtypeStruct((_B, _NCLS), jnp.float32),
)


def kernel(token_ids, emb, W1, b1, g1, be1, W2, b2, g2, be2, Wout, bout):
    tok = token_ids.astype(jnp.int32)
    tokp = jnp.pad(tok, ((0, 0), (0, _LP - _L)))
    pooled_sum = _sc_pool(tokp, emb)
    return _mlp(
        pooled_sum,
        W1, b1.reshape(1, -1), g1.reshape(1, -1), be1.reshape(1, -1),
        W2, b2.reshape(1, -1), g2.reshape(1, -1), be2.reshape(1, -1),
        Wout, bout.reshape(1, -1),
    )


# speedup vs baseline: 9.9492x; 9.9492x over previous
import functools

import jax
import jax.numpy as jnp
from jax import lax
from jax.experimental import pallas as pl
from jax.experimental.pallas import tpu as pltpu
from jax.experimental.pallas import tpu_sc as plsc

_B = 4096
_L = 50
_LP = 56
_D = 128
_H1 = 256
_H2 = 128
_NCLS = 50
_EPS = 1e-5

_NC = 2
_NS = 16
_NW = _NC * _NS
_EPW = _B // _NW
_PAIRS = _EPW // 2
_PW = 2 * _LP
_LANES = 16
_VPR = _D // _LANES


_NBUF = 3
_CH = 128
_PPT = _B * _L // _NW
_NCHUNK = _PPT // _CH
_BH = _B // _NC


def _sc_pool_body(tok_hbm, dst_hbm, emb_hbm, out_hbm,
                  tok_v, dst_v, rows, pooled_sp, *sems):
    c = lax.axis_index("c")
    s = lax.axis_index("s")
    w = c * _NS + s

    pltpu.sync_copy(tok_hbm.at[w], tok_v)
    pltpu.sync_copy(dst_hbm.at[w], dst_v)

    base = c * _BH

    def _localize(k, carry):
        for v in range(_CH // _LANES):
            sl = pl.ds(v * _LANES, _LANES)
            dst_v[k, sl] = dst_v[k, sl] - base
        return carry

    lax.fori_loop(0, _NCHUNK, _localize, 0)

    zrows = _BH // _NS

    def _zero(r, carry):
        for v in range(_VPR):
            rows[0, r, pl.ds(v * _LANES, _LANES)] = jnp.zeros(
                (_LANES,), jnp.float32)
        return carry

    lax.fori_loop(0, zrows, _zero, 0)
    pltpu.sync_copy(rows.at[0, pl.ds(0, zrows)],
                    pooled_sp.at[pl.ds(s * zrows, zrows)])
    plsc.subcore_barrier()

    def fire(k, b):
        pltpu.make_async_copy(
            emb_hbm.at[tok_v.at[k]], rows.at[b], sems[b]
        ).start()

    def wait(b):
        pltpu.make_async_copy(
            emb_hbm.at[pl.ds(0, _CH)], rows.at[b], sems[b]
        ).wait()

    for b in range(_NBUF):
        fire(b, b)

    def step(j, carry):
        k0 = j * _NBUF
        for b in range(_NBUF):
            k = k0 + b

            @pl.when(k < _NCHUNK)
            def _():
                wait(b)
                pltpu.sync_copy(rows.at[b], pooled_sp.at[dst_v.at[k]],
                                add=True)

                @pl.when(k + _NBUF < _NCHUNK)
                def _():
                    fire(k + _NBUF, b)
        return carry

    lax.fori_loop(0, (_NCHUNK + _NBUF - 1) // _NBUF, step, 0)

    plsc.subcore_barrier()
    pltpu.sync_copy(pooled_sp.at[pl.ds(s * zrows, zrows)],
                    out_hbm.at[pl.ds(c * _BH + s * zrows, zrows)])


_sc_pool = functools.partial(
    pl.kernel,
    out_type=jax.ShapeDtypeStruct((_B, _D), jnp.float32),
    mesh=plsc.VectorSubcoreMesh(core_axis_name="c", subcore_axis_name="s"),
    scratch_types=[
        pltpu.VMEM((_NCHUNK, _CH), jnp.int32),
        pltpu.VMEM((_NCHUNK, _CH), jnp.int32),
        pltpu.VMEM((_NBUF, _CH, _D), jnp.float32),
        pltpu.VMEM_SHARED((_BH, _D), jnp.float32),
    ] + [pltpu.SemaphoreType.DMA] * _NBUF,
)(_sc_pool_body)


def _mlp_body(ps, w1, b1, g1, be1, w2, b2, g2, be2, wout, bout, out):
    x = ps[...] * (1.0 / _L)

    h = lax.dot_general(x, w1[...], (((1,), (1,)), ((), ())),
                        preferred_element_type=jnp.float32) + b1[...]
    mu = jnp.mean(h, axis=0, keepdims=True)
    d = h - mu
    var = jnp.mean(d * d, axis=0, keepdims=True)
    h = g1[...] * d / jnp.sqrt(var + _EPS) + be1[...]
    h = jnp.maximum(h, 0.0)

    h = lax.dot_general(h, w2[...], (((1,), (1,)), ((), ())),
                        preferred_element_type=jnp.float32) + b2[...]
    mu = jnp.mean(h, axis=0, keepdims=True)
    d = h - mu
    var = jnp.mean(d * d, axis=0, keepdims=True)
    h = g2[...] * d / jnp.sqrt(var + _EPS) + be2[...]
    h = jnp.maximum(h, 0.0)

    out[...] = lax.dot_general(h, wout[...], (((1,), (1,)), ((), ())),
                               preferred_element_type=jnp.float32) + bout[...]


_mlp = pl.pallas_call(
    _mlp_body,
    out_shape=jax.ShapeDtypeStruct((_B, _NCLS), jnp.float32),
)


def kernel(token_ids, emb, W1, b1, g1, be1, W2, b2, g2, be2, Wout, bout):
    tok3 = token_ids.astype(jnp.int32).reshape(_NW, _NCHUNK, _CH)
    dst3 = jnp.repeat(
        jnp.arange(_B, dtype=jnp.int32), _L).reshape(_NW, _NCHUNK, _CH)
    pooled_sum = _sc_pool(tok3, dst3, emb)
    return _mlp(
        pooled_sum,
        W1, b1.reshape(1, -1), g1.reshape(1, -1), be1.reshape(1, -1),
        W2, b2.reshape(1, -1), g2.reshape(1, -1), be2.reshape(1, -1),
        Wout, bout.reshape(1, -1),
    )


# speedup vs baseline: 9.9699x; 1.0021x over previous
import functools

import jax
import jax.numpy as jnp
from jax import lax
from jax.experimental import pallas as pl
from jax.experimental.pallas import tpu as pltpu
from jax.experimental.pallas import tpu_sc as plsc

_B = 4096
_L = 50
_LP = 56
_D = 128
_H1 = 256
_H2 = 128
_NCLS = 50
_EPS = 1e-5

_NC = 2
_NS = 16
_NW = _NC * _NS
_EPW = _B // _NW
_PAIRS = _EPW // 2
_PW = 2 * _LP
_LANES = 16
_VPR = _D // _LANES


_NBUF = 3
_CH = 128
_PPT = _B * _L // _NW
_NCHUNK = _PPT // _CH
_BH = _B // _NC


def _sc_pool_body(tok_hbm, dst_hbm, emb_hbm, out_hbm,
                  tok_v, dst_v, rows, pooled_sp, *sems):
    c = lax.axis_index("c")
    s = lax.axis_index("s")
    w = c * _NS + s

    pltpu.sync_copy(tok_hbm.at[w], tok_v)
    pltpu.sync_copy(dst_hbm.at[w], dst_v)

    zrows = _BH // _NS

    def _zero(r, carry):
        for v in range(_VPR):
            rows[0, r, pl.ds(v * _LANES, _LANES)] = jnp.zeros(
                (_LANES,), jnp.float32)
        return carry

    lax.fori_loop(0, zrows, _zero, 0)
    pltpu.sync_copy(rows.at[0, pl.ds(0, zrows)],
                    pooled_sp.at[pl.ds(s * zrows, zrows)])
    plsc.subcore_barrier()

    def fire(k, b):
        pltpu.make_async_copy(
            emb_hbm.at[tok_v.at[k]], rows.at[b], sems[b]
        ).start()

    def wait(b):
        pltpu.make_async_copy(
            emb_hbm.at[pl.ds(0, _CH)], rows.at[b], sems[b]
        ).wait()

    for b in range(_NBUF):
        fire(b, b)

    def step(j, carry):
        k0 = j * _NBUF
        for b in range(_NBUF):
            k = k0 + b

            @pl.when(k < _NCHUNK)
            def _():
                wait(b)
                pltpu.sync_copy(rows.at[b], pooled_sp.at[dst_v.at[k]],
                                add=True)

                @pl.when(k + _NBUF < _NCHUNK)
                def _():
                    fire(k + _NBUF, b)
        return carry

    lax.fori_loop(0, (_NCHUNK + _NBUF - 1) // _NBUF, step, 0)

    plsc.subcore_barrier()
    pltpu.sync_copy(pooled_sp.at[pl.ds(s * zrows, zrows)],
                    out_hbm.at[pl.ds(c * _BH + s * zrows, zrows)])


_sc_pool = functools.partial(
    pl.kernel,
    out_type=jax.ShapeDtypeStruct((_B, _D), jnp.float32),
    mesh=plsc.VectorSubcoreMesh(core_axis_name="c", subcore_axis_name="s"),
    scratch_types=[
        pltpu.VMEM((_NCHUNK, _CH), jnp.int32),
        pltpu.VMEM((_NCHUNK, _CH), jnp.int32),
        pltpu.VMEM((_NBUF, _CH, _D), jnp.float32),
        pltpu.VMEM_SHARED((_BH, _D), jnp.float32),
    ] + [pltpu.SemaphoreType.DMA] * _NBUF,
)(_sc_pool_body)


def _mlp_body(ps, w1, b1, g1, be1, w2, b2, g2, be2, wout, bout, out):
    x = ps[...] * (1.0 / _L)

    h = lax.dot_general(x, w1[...], (((1,), (1,)), ((), ())),
                        preferred_element_type=jnp.float32) + b1[...]
    mu = jnp.mean(h, axis=0, keepdims=True)
    d = h - mu
    var = jnp.mean(d * d, axis=0, keepdims=True)
    h = g1[...] * d / jnp.sqrt(var + _EPS) + be1[...]
    h = jnp.maximum(h, 0.0)

    h = lax.dot_general(h, w2[...], (((1,), (1,)), ((), ())),
                        preferred_element_type=jnp.float32) + b2[...]
    mu = jnp.mean(h, axis=0, keepdims=True)
    d = h - mu
    var = jnp.mean(d * d, axis=0, keepdims=True)
    h = g2[...] * d / jnp.sqrt(var + _EPS) + be2[...]
    h = jnp.maximum(h, 0.0)

    out[...] = lax.dot_general(h, wout[...], (((1,), (1,)), ((), ())),
                               preferred_element_type=jnp.float32) + bout[...]


_mlp = pl.pallas_call(
    _mlp_body,
    out_shape=jax.ShapeDtypeStruct((_B, _NCLS), jnp.float32),
)


def kernel(token_ids, emb, W1, b1, g1, be1, W2, b2, g2, be2, Wout, bout):
    tok3 = token_ids.astype(jnp.int32).reshape(_NW, _NCHUNK, _CH)
    dst3 = jnp.repeat(jnp.arange(_B, dtype=jnp.int32) % _BH,
                      _L).reshape(_NW, _NCHUNK, _CH)
    pooled_sum = _sc_pool(tok3, dst3, emb)
    return _mlp(
        pooled_sum,
        W1, b1.reshape(1, -1), g1.reshape(1, -1), be1.reshape(1, -1),
        W2, b2.reshape(1, -1), g2.reshape(1, -1), be2.reshape(1, -1),
        Wout, bout.reshape(1, -1),
    )


# speedup vs baseline: 11.7801x; 1.1816x over previous
import functools

import jax
import jax.numpy as jnp
import numpy as np
from jax import lax
from jax.experimental import pallas as pl
from jax.experimental.pallas import tpu as pltpu
from jax.experimental.pallas import tpu_sc as plsc

_B = 4096
_L = 50
_LP = 56
_D = 128
_H1 = 256
_H2 = 128
_NCLS = 50
_EPS = 1e-5

_NC = 2
_NS = 16
_NW = _NC * _NS
_EPW = _B // _NW
_PAIRS = _EPW // 2
_PW = 2 * _LP
_LANES = 16
_VPR = _D // _LANES


_NBUF = 4
_CH = 128
_PPT = _B * _L // _NW
_NCHUNK = _PPT // _CH
_BH = _B // _NC


def _sc_pool_body(tok_hbm, dst_hbm, emb_hbm, out_hbm,
                  tok_v, dst_v, rows, pooled_sp, *sems):
    c = lax.axis_index("c")
    s = lax.axis_index("s")
    w = c * _NS + s

    pltpu.sync_copy(tok_hbm.at[w], tok_v)
    pltpu.sync_copy(dst_hbm.at[w], dst_v)

    zrows = _BH // _NS

    def _zero(r, carry):
        for v in range(_VPR):
            rows[0, r, pl.ds(v * _LANES, _LANES)] = jnp.zeros(
                (_LANES,), jnp.float32)
        return carry

    lax.fori_loop(0, zrows, _zero, 0)
    pltpu.sync_copy(rows.at[0, pl.ds(0, zrows)],
                    pooled_sp.at[pl.ds(s * zrows, zrows)])
    plsc.subcore_barrier()

    gsems = sems[:_NBUF]
    ssems = sems[_NBUF:]

    def fire(k, b):
        pltpu.make_async_copy(
            emb_hbm.at[tok_v.at[k]], rows.at[b], gsems[b]
        ).start()

    def wait_gather(b):
        pltpu.make_async_copy(
            emb_hbm.at[pl.ds(0, _CH)], rows.at[b], gsems[b]
        ).wait()

    def scatter_start(k, b):
        pltpu.make_async_copy(
            rows.at[b], pooled_sp.at[dst_v.at[k]], ssems[b]
        ).start()

    def wait_scatter(b):
        pltpu.make_async_copy(
            rows.at[b], pooled_sp.at[dst_v.at[0]], ssems[b]
        ).wait()

    for b in range(_NBUF):
        fire(b, b)

    def step(j, carry):
        k0 = j * _NBUF
        for b in range(_NBUF):
            k = k0 + b
            bprev = (b - 1) % _NBUF

            @pl.when(k < _NCHUNK)
            def _():
                wait_gather(b)
                scatter_start(k, b)

                @pl.when((k >= 1) & (k - 1 + _NBUF < _NCHUNK))
                def _():
                    wait_scatter(bprev)
                    fire(k - 1 + _NBUF, bprev)
        return carry

    lax.fori_loop(0, (_NCHUNK + _NBUF - 1) // _NBUF, step, 0)

    for b in range(_NBUF):
        wait_scatter(b)

    plsc.subcore_barrier()
    pltpu.sync_copy(pooled_sp.at[pl.ds(s * zrows, zrows)],
                    out_hbm.at[pl.ds(c * _BH + s * zrows, zrows)])


_sc_pool = functools.partial(
    pl.kernel,
    out_type=jax.ShapeDtypeStruct((_B, _D), jnp.float32),
    mesh=plsc.VectorSubcoreMesh(core_axis_name="c", subcore_axis_name="s"),
    scratch_types=[
        pltpu.VMEM((_NCHUNK, _CH), jnp.int32),
        pltpu.VMEM((_NCHUNK, _CH), jnp.int32),
        pltpu.VMEM((_NBUF, _CH, _D), jnp.float32),
        pltpu.VMEM_SHARED((_BH, _D), jnp.float32),
    ] + [pltpu.SemaphoreType.DMA] * (2 * _NBUF),
)(_sc_pool_body)

_DST3 = np.repeat(np.arange(_B, dtype=np.int32) % _BH,
                  _L).reshape(_NW, _NCHUNK, _CH)


def _mlp_body(ps, w1, b1, g1, be1, w2, b2, g2, be2, wout, bout, out):
    x = ps[...] * (1.0 / _L)

    h = lax.dot_general(x, w1[...], (((1,), (1,)), ((), ())),
                        preferred_element_type=jnp.float32) + b1[...]
    mu = jnp.mean(h, axis=0, keepdims=True)
    d = h - mu
    var = jnp.mean(d * d, axis=0, keepdims=True)
    h = g1[...] * d / jnp.sqrt(var + _EPS) + be1[...]
    h = jnp.maximum(h, 0.0)

    h = lax.dot_general(h, w2[...], (((1,), (1,)), ((), ())),
                        preferred_element_type=jnp.float32) + b2[...]
    mu = jnp.mean(h, axis=0, keepdims=True)
    d = h - mu
    var = jnp.mean(d * d, axis=0, keepdims=True)
    h = g2[...] * d / jnp.sqrt(var + _EPS) + be2[...]
    h = jnp.maximum(h, 0.0)

    out[...] = lax.dot_general(h, wout[...], (((1,), (1,)), ((), ())),
                               preferred_element_type=jnp.float32) + bout[...]


_mlp = pl.pallas_call(
    _mlp_body,
    out_shape=jax.ShapeDtypeStruct((_B, _NCLS), jnp.float32),
)


def kernel(token_ids, emb, W1, b1, g1, be1, W2, b2, g2, be2, Wout, bout):
    tok3 = token_ids.astype(jnp.int32).reshape(_NW, _NCHUNK, _CH)
    pooled_sum = _sc_pool(tok3, _DST3, emb)
    return _mlp(
        pooled_sum,
        W1, b1.reshape(1, -1), g1.reshape(1, -1), be1.reshape(1, -1),
        W2, b2.reshape(1, -1), g2.reshape(1, -1), be2.reshape(1, -1),
        Wout, bout.reshape(1, -1),
    )
